# Initial kernel scaffold; baseline (speedup 1.0000x reference)
import jax
import jax.numpy as jnp
from jax.experimental import pallas as pl


def kernel(features, positive_edges, negative_edges, non_zero_index, non_zero_value, W):
    raise NotImplementedError("write your pallas kernel here")


# speedup vs baseline: 3.0251x; 3.0251x over previous
import functools

import jax
import jax.numpy as jnp
from jax import lax
from jax.experimental import pallas as pl
from jax.experimental.pallas import tpu as pltpu
from jax.experimental.pallas import tpu_sc as plsc

N_NODES = 10000
D = 128
NNZ = 320000
NPOS = 100000
NNEG = 100000

NC = 2
NS = 16
NW = NC * NS

N_PAD = 10240
ROWS_PT = N_PAD // NS

_mesh = plsc.VectorSubcoreMesh(
    core_axis_name="c", subcore_axis_name="s", num_cores=NC, num_subcores=NS
)

NH = D // 2
SUB_A = 80
RPC_A = 8
KA = RPC_A * SUB_A
CHUNKS_A = NNZ // KA
ITERS_A = (CHUNKS_A + NS - 1) // NS


@functools.partial(
    pl.kernel,
    out_type=jax.ShapeDtypeStruct((NC, N_PAD, NH), jnp.float32),
    mesh=_mesh,
    scratch_types=[
        pltpu.VMEM((RPC_A, SUB_A), jnp.int32),
        pltpu.VMEM((RPC_A, SUB_A), jnp.int32),
        pltpu.VMEM((KA,), jnp.float32),
        pltpu.VMEM((RPC_A, SUB_A, NH), jnp.float32),
        pltpu.VMEM_SHARED((N_PAD, NH), jnp.float32),
        pltpu.SemaphoreType.DMA,
    ],
    compiler_params=pltpu.CompilerParams(needs_layout_passes=False, use_tc_tiling_on_sc=False),
)
def _gconv_agg(feat2, src2, dst2, vals, zeros, out, idx_s, idx_d, vals_v, rows_v, agg_sh, sem):
    cid = lax.axis_index("c")
    sid = lax.axis_index("s")

    pltpu.sync_copy(
        zeros.at[pl.ds(sid * ROWS_PT, ROWS_PT)],
        agg_sh.at[pl.ds(sid * ROWS_PT, ROWS_PT)],
    )
    plsc.subcore_barrier()
    src_off = cid * N_NODES

    def chunk(it, carry):
        ch = it * NS + sid

        @pl.when(ch < CHUNKS_A)
        def _():
            row0 = ch * RPC_A
            ebase = ch * KA
            pltpu.sync_copy(src2.at[pl.ds(row0, RPC_A)], idx_s)
            pltpu.sync_copy(dst2.at[pl.ds(row0, RPC_A)], idx_d)
            pltpu.sync_copy(vals.at[pl.ds(ebase, KA)], vals_v)
            for i in range(RPC_A):
                for g in range(SUB_A // 16):
                    sl = pl.ds(g * 16, 16)
                    idx_s[i, sl] = idx_s[i, sl] + src_off
            descs = [
                pltpu.async_copy(feat2.at[idx_s.at[i]], rows_v.at[i], sem)
                for i in range(RPC_A)
            ]
            for dsc in descs:
                dsc.wait()

            def sub(i, c1):
                def sgroup(g, c2):
                    for r in range(16):
                        row = g * 16 + r
                        bc = plsc.load_gather(
                            vals_v,
                            [jnp.full((16,), i * SUB_A + row, jnp.int32)],
                        )
                        for j in range(NH // 16):
                            rows_v[i, row, pl.ds(j * 16, 16)] = (
                                rows_v[i, row, pl.ds(j * 16, 16)] * bc
                            )
                    return c2

                lax.fori_loop(0, SUB_A // 16, sgroup, 0)
                return c1

            lax.fori_loop(0, RPC_A, sub, 0)

            for i in range(RPC_A):
                pltpu.sync_copy(rows_v.at[i], agg_sh.at[idx_d.at[i]], add=True)

        return carry

    lax.fori_loop(0, ITERS_A, chunk, 0)
    plsc.subcore_barrier()
    pltpu.sync_copy(
        agg_sh.at[pl.ds(sid * ROWS_PT, ROWS_PT)],
        out.at[cid, pl.ds(sid * ROWS_PT, ROWS_PT)],
    )


BR = 2000


def _mm_body(a_ref, w_ref, o_ref):
    xl = a_ref[0]
    xr = a_ref[1]
    acc = jnp.dot(xl, w_ref[:NH, :], preferred_element_type=jnp.float32)
    acc = acc + jnp.dot(xr, w_ref[NH:, :], preferred_element_type=jnp.float32)
    o_ref[...] = jnp.maximum(acc, 0.0)


def _matmul_relu(agg_parts, W):
    return pl.pallas_call(
        _mm_body,
        grid=(N_NODES // BR,),
        in_specs=[
            pl.BlockSpec((NC, BR, NH), lambda i: (0, i, 0)),
            pl.BlockSpec((D, D), lambda i: (0, 0)),
        ],
        out_specs=pl.BlockSpec((BR, D), lambda i: (i, 0)),
        out_shape=jax.ShapeDtypeStruct((N_NODES, D), jnp.float32),
    )(agg_parts, W)


SUB_C = 40
RPC_C = 8
KC = RPC_C * SUB_C
HALF_PAD = 102400
NE_TOT = 2 * HALF_PAD
CHUNKS_C = NE_TOT // KC
ITERS_C = CHUNKS_C // NW


@functools.partial(
    pl.kernel,
    out_type=jax.ShapeDtypeStruct((NE_TOT,), jnp.float32),
    mesh=_mesh,
    scratch_types=[
        pltpu.VMEM((RPC_C, SUB_C), jnp.int32),
        pltpu.VMEM((RPC_C, SUB_C), jnp.int32),
        pltpu.VMEM((RPC_C, SUB_C, D), jnp.float32),
        pltpu.VMEM((RPC_C, SUB_C, D), jnp.float32),
        pltpu.VMEM((16, 16), jnp.float32),
        pltpu.VMEM((KC,), jnp.float32),
        pltpu.SemaphoreType.DMA,
    ],
    compiler_params=pltpu.CompilerParams(needs_layout_passes=False, use_tc_tiling_on_sc=False),
)
def _scores(emb, e0, e1, out, idx_a, idx_b, rows_a, rows_b, stage_v, sc_v, sem):
    cid = lax.axis_index("c")
    sid = lax.axis_index("s")
    wid = sid * NC + cid
    lanes = jnp.arange(16, dtype=jnp.int32)

    def it(i, carry):
        ch = i * NW + wid
        row0 = ch * RPC_C
        pltpu.sync_copy(e0.at[pl.ds(row0, RPC_C)], idx_a)
        pltpu.sync_copy(e1.at[pl.ds(row0, RPC_C)], idx_b)
        descs = [
            pltpu.async_copy(emb.at[idx_a.at[b]], rows_a.at[b], sem)
            for b in range(RPC_C)
        ] + [
            pltpu.async_copy(emb.at[idx_b.at[b]], rows_b.at[b], sem)
            for b in range(RPC_C)
        ]
        for dsc in descs:
            dsc.wait()

        def grp(g, c2):
            for k in range(16):
                e = g * 16 + k
                i2 = e // SUB_C
                r = e - i2 * SUB_C
                p = rows_a[i2, r, pl.ds(0, 16)] * rows_b[i2, r, pl.ds(0, 16)]
                for j in range(1, D // 16):
                    p = p + (
                        rows_a[i2, r, pl.ds(j * 16, 16)]
                        * rows_b[i2, r, pl.ds(j * 16, 16)]
                    )
                stage_v[k, :] = p
            acc = plsc.load_gather(stage_v, [lanes, jnp.zeros((16,), jnp.int32)])
            for j in range(1, 16):
                acc = acc + plsc.load_gather(
                    stage_v, [lanes, jnp.full((16,), j, jnp.int32)]
                )
            sc_v[pl.ds(g * 16, 16)] = acc
            return c2

        lax.fori_loop(0, KC // 16, grp, 0)
        pltpu.sync_copy(sc_v, out.at[pl.ds(ch * KC, KC)])
        return carry

    lax.fori_loop(0, ITERS_C, it, 0)


def _loss_body(p_ref, n_ref, o_ref):
    t = jax.nn.softplus(-p_ref[...]) + jax.nn.softplus(n_ref[...])
    o_ref[0, 0] = jnp.sum(t) / NPOS


def _loss(pos_sc, neg_sc):
    return pl.pallas_call(
        _loss_body,
        in_specs=[
            pl.BlockSpec((625, 160), lambda: (0, 0)),
            pl.BlockSpec((625, 160), lambda: (0, 0)),
        ],
        out_specs=pl.BlockSpec(memory_space=pltpu.SMEM),
        out_shape=jax.ShapeDtypeStruct((1, 1), jnp.float32),
    )(pos_sc, neg_sc)


def _pad_col(col):
    return jnp.concatenate([col, jnp.zeros((HALF_PAD - NPOS,), jnp.int32)])


def kernel(features, positive_edges, negative_edges, non_zero_index, non_zero_value, W):
    src2 = non_zero_index[0].reshape(NNZ // SUB_A, SUB_A)
    dst2 = non_zero_index[1].reshape(NNZ // SUB_A, SUB_A)
    zeros = jnp.zeros((N_PAD, NH), jnp.float32)
    feat2 = jnp.concatenate([features[:, :NH], features[:, NH:]], axis=0)
    agg_parts = _gconv_agg(feat2, src2, dst2, non_zero_value, zeros)
    emb = _matmul_relu(agg_parts, W)
    e0 = jnp.concatenate(
        [_pad_col(positive_edges[:, 0]), _pad_col(negative_edges[:, 0])]
    ).reshape(NE_TOT // SUB_C, SUB_C)
    e1 = jnp.concatenate(
        [_pad_col(positive_edges[:, 1]), _pad_col(negative_edges[:, 1])]
    ).reshape(NE_TOT // SUB_C, SUB_C)
    flat = _scores(emb, e0, e1)
    pos_sc = flat[:NPOS].reshape(625, 160)
    neg_sc = flat[HALF_PAD : HALF_PAD + NNEG].reshape(625, 160)
    return _loss(pos_sc, neg_sc)[0, 0]


# speedup vs baseline: 3.8151x; 1.2612x over previous
import functools

import jax
import jax.numpy as jnp
from jax import lax
from jax.experimental import pallas as pl
from jax.experimental.pallas import tpu as pltpu
from jax.experimental.pallas import tpu_sc as plsc

N_NODES = 10000
D = 128
NNZ = 320000
NPOS = 100000
NNEG = 100000

NC = 2
NS = 16
NW = NC * NS

N_PAD = 10240
ROWS_PT = N_PAD // NS

_mesh = plsc.VectorSubcoreMesh(
    core_axis_name="c", subcore_axis_name="s", num_cores=NC, num_subcores=NS
)

NH = D // 2
SUB_A = 80
RPC_A = 8
KA = RPC_A * SUB_A
CHUNKS_A = NNZ // KA
ITERS_A = (CHUNKS_A + NS - 1) // NS


@functools.partial(
    pl.kernel,
    out_type=jax.ShapeDtypeStruct((NC, N_PAD, NH), jnp.float32),
    mesh=_mesh,
    scratch_types=[
        pltpu.VMEM((RPC_A, SUB_A), jnp.int32),
        pltpu.VMEM((RPC_A, SUB_A), jnp.int32),
        pltpu.VMEM((KA,), jnp.float32),
        pltpu.VMEM((RPC_A, SUB_A, NH), jnp.float32),
        pltpu.VMEM_SHARED((N_PAD, NH), jnp.float32),
        pltpu.SemaphoreType.DMA,
    ],
    compiler_params=pltpu.CompilerParams(needs_layout_passes=False, use_tc_tiling_on_sc=False),
)
def _gconv_agg(feat2, src2, dst2, vals, zeros, out, idx_s, idx_d, vals_v, rows_v, agg_sh, sem):
    cid = lax.axis_index("c")
    sid = lax.axis_index("s")

    pltpu.sync_copy(
        zeros.at[pl.ds(sid * ROWS_PT, ROWS_PT)],
        agg_sh.at[pl.ds(sid * ROWS_PT, ROWS_PT)],
    )
    plsc.subcore_barrier()
    src_off = cid * N_NODES

    def chunk(it, carry):
        ch = it * NS + sid

        @pl.when(ch < CHUNKS_A)
        def _():
            row0 = ch * RPC_A
            ebase = ch * KA
            pltpu.sync_copy(src2.at[pl.ds(row0, RPC_A)], idx_s)
            pltpu.sync_copy(dst2.at[pl.ds(row0, RPC_A)], idx_d)
            pltpu.sync_copy(vals.at[pl.ds(ebase, KA)], vals_v)
            for i in range(RPC_A):
                for g in range(SUB_A // 16):
                    sl = pl.ds(g * 16, 16)
                    idx_s[i, sl] = idx_s[i, sl] + src_off
            descs = [
                pltpu.async_copy(feat2.at[idx_s.at[i]], rows_v.at[i], sem)
                for i in range(RPC_A)
            ]
            for dsc in descs:
                dsc.wait()

            def sub(i, c1):
                def sgroup(g, c2):
                    for r in range(16):
                        row = g * 16 + r
                        bc = plsc.load_gather(
                            vals_v,
                            [jnp.full((16,), i * SUB_A + row, jnp.int32)],
                        )
                        for j in range(NH // 16):
                            rows_v[i, row, pl.ds(j * 16, 16)] = (
                                rows_v[i, row, pl.ds(j * 16, 16)] * bc
                            )
                    return c2

                lax.fori_loop(0, SUB_A // 16, sgroup, 0)
                return c1

            lax.fori_loop(0, RPC_A, sub, 0)

            for i in range(RPC_A):
                pltpu.sync_copy(rows_v.at[i], agg_sh.at[idx_d.at[i]], add=True)

        return carry

    lax.fori_loop(0, ITERS_A, chunk, 0)
    plsc.subcore_barrier()
    pltpu.sync_copy(
        agg_sh.at[pl.ds(sid * ROWS_PT, ROWS_PT)],
        out.at[cid, pl.ds(sid * ROWS_PT, ROWS_PT)],
    )


BR = 2000


def _mm_body(a_ref, w_ref, o_ref):
    xl = a_ref[0]
    xr = a_ref[1]
    acc = jnp.dot(xl, w_ref[:NH, :], preferred_element_type=jnp.float32)
    acc = acc + jnp.dot(xr, w_ref[NH:, :], preferred_element_type=jnp.float32)
    o_ref[...] = jnp.maximum(acc, 0.0).astype(jnp.bfloat16)


def _matmul_relu(agg_parts, W):
    return pl.pallas_call(
        _mm_body,
        grid=(N_NODES // BR,),
        in_specs=[
            pl.BlockSpec((NC, BR, NH), lambda i: (0, i, 0)),
            pl.BlockSpec((D, D), lambda i: (0, 0)),
        ],
        out_specs=pl.BlockSpec((BR, D), lambda i: (i, 0)),
        out_shape=jax.ShapeDtypeStruct((N_NODES, D), jnp.bfloat16),
    )(agg_parts, W)


SUB_C = 40
RPC_C = 8
KC = RPC_C * SUB_C
HALF_PAD = 102400
NE_TOT = 2 * HALF_PAD
CHUNKS_C = NE_TOT // KC
ITERS_C = CHUNKS_C // NW


@functools.partial(
    pl.kernel,
    out_type=jax.ShapeDtypeStruct((NE_TOT,), jnp.float32),
    mesh=_mesh,
    scratch_types=[
        pltpu.VMEM((RPC_C, SUB_C), jnp.int32),
        pltpu.VMEM((RPC_C, SUB_C), jnp.int32),
        pltpu.VMEM((RPC_C, SUB_C, D), jnp.bfloat16),
        pltpu.VMEM((RPC_C, SUB_C, D), jnp.bfloat16),
        pltpu.VMEM((16, 16), jnp.float32),
        pltpu.VMEM((KC,), jnp.float32),
        pltpu.SemaphoreType.DMA,
    ],
    compiler_params=pltpu.CompilerParams(needs_layout_passes=False, use_tc_tiling_on_sc=False),
)
def _scores(emb, e0, e1, out, idx_a, idx_b, rows_a, rows_b, stage_v, sc_v, sem):
    cid = lax.axis_index("c")
    sid = lax.axis_index("s")
    wid = sid * NC + cid
    lanes = jnp.arange(16, dtype=jnp.int32)

    def it(i, carry):
        ch = i * NW + wid
        row0 = ch * RPC_C
        pltpu.sync_copy(e0.at[pl.ds(row0, RPC_C)], idx_a)
        pltpu.sync_copy(e1.at[pl.ds(row0, RPC_C)], idx_b)
        descs = [
            pltpu.async_copy(emb.at[idx_a.at[b]], rows_a.at[b], sem)
            for b in range(RPC_C)
        ] + [
            pltpu.async_copy(emb.at[idx_b.at[b]], rows_b.at[b], sem)
            for b in range(RPC_C)
        ]
        for dsc in descs:
            dsc.wait()

        def grp(g, c2):
            for k in range(16):
                e = g * 16 + k
                i2 = e // SUB_C
                r = e - i2 * SUB_C
                p = None
                for j in range(D // 32):
                    a0, a1 = plsc.unpack(
                        rows_a[i2, r, pl.ds(j * 32, 32)],
                        format=plsc.PackFormat.INTERLEAVED,
                    )
                    b0, b1 = plsc.unpack(
                        rows_b[i2, r, pl.ds(j * 32, 32)],
                        format=plsc.PackFormat.INTERLEAVED,
                    )
                    t = a0 * b0 + a1 * b1
                    p = t if p is None else p + t
                stage_v[k, :] = p
            acc = plsc.load_gather(stage_v, [lanes, jnp.zeros((16,), jnp.int32)])
            for j in range(1, 16):
                acc = acc + plsc.load_gather(
                    stage_v, [lanes, jnp.full((16,), j, jnp.int32)]
                )
            sc_v[pl.ds(g * 16, 16)] = acc
            return c2

        lax.fori_loop(0, KC // 16, grp, 0)
        pltpu.sync_copy(sc_v, out.at[pl.ds(ch * KC, KC)])
        return carry

    lax.fori_loop(0, ITERS_C, it, 0)


def _loss_body(p_ref, n_ref, o_ref):
    t = jax.nn.softplus(-p_ref[...]) + jax.nn.softplus(n_ref[...])
    o_ref[0, 0] = jnp.sum(t) / NPOS


def _loss(pos_sc, neg_sc):
    return pl.pallas_call(
        _loss_body,
        in_specs=[
            pl.BlockSpec((625, 160), lambda: (0, 0)),
            pl.BlockSpec((625, 160), lambda: (0, 0)),
        ],
        out_specs=pl.BlockSpec(memory_space=pltpu.SMEM),
        out_shape=jax.ShapeDtypeStruct((1, 1), jnp.float32),
    )(pos_sc, neg_sc)


def _pad_col(col):
    return jnp.concatenate([col, jnp.zeros((HALF_PAD - NPOS,), jnp.int32)])


def kernel(features, positive_edges, negative_edges, non_zero_index, non_zero_value, W):
    src2 = non_zero_index[0].reshape(NNZ // SUB_A, SUB_A)
    dst2 = non_zero_index[1].reshape(NNZ // SUB_A, SUB_A)
    zeros = jnp.zeros((N_PAD, NH), jnp.float32)
    feat2 = jnp.concatenate([features[:, :NH], features[:, NH:]], axis=0)
    agg_parts = _gconv_agg(feat2, src2, dst2, non_zero_value, zeros)
    emb = _matmul_relu(agg_parts, W)
    e0 = jnp.concatenate(
        [_pad_col(positive_edges[:, 0]), _pad_col(negative_edges[:, 0])]
    ).reshape(NE_TOT // SUB_C, SUB_C)
    e1 = jnp.concatenate(
        [_pad_col(positive_edges[:, 1]), _pad_col(negative_edges[:, 1])]
    ).reshape(NE_TOT // SUB_C, SUB_C)
    flat = _scores(emb, e0, e1)
    pos_sc = flat[:NPOS].reshape(625, 160)
    neg_sc = flat[HALF_PAD : HALF_PAD + NNEG].reshape(625, 160)
    return _loss(pos_sc, neg_sc)[0, 0]
